# baseline (speedup 1.0000x reference)
import functools

import jax
import jax.numpy as jnp
from jax import lax
from jax.experimental import pallas as pl
from jax.experimental.pallas import tpu as pltpu
from jax.experimental.pallas import tpu_sc as plsc

NC = 2
NS = 16
L = 16
NW = NC * NS


@functools.cache
def _build(B: int):
    BPW = B // NW
    CH = BPW // L

    mesh = plsc.VectorSubcoreMesh(core_axis_name="c", subcore_axis_name="s",
                                  num_cores=NC, num_subcores=NS)

    @functools.partial(
        pl.kernel,
        out_type=jax.ShapeDtypeStruct((B, 6), jnp.float32),
        mesh=mesh,
        scratch_types=[
            pltpu.VMEM((BPW, 4), jnp.float32),
            pltpu.VMEM((BPW, 4), jnp.int32),
            pltpu.VMEM((BPW, 6), jnp.float32),
            pltpu.VMEM((64,), jnp.float32),
            pltpu.VMEM((6, 16), jnp.float32),
            pltpu.VMEM((6, 64), jnp.float32),
        ],
        compiler_params=pltpu.CompilerParams(
            needs_layout_passes=False, use_tc_tiling_on_sc=False),
    )
    def sc_encoder(cont_hbm, idx_hbm, embf_hbm, wb_hbm, out_hbm,
                   cont_v, idx_v, out_v, embf_v, wb_v, tab_v):
        wid = lax.axis_index("s") * NC + lax.axis_index("c")
        base = wid * BPW

        pltpu.sync_copy(cont_hbm.at[pl.ds(base, BPW)], cont_v)
        pltpu.sync_copy(idx_hbm.at[pl.ds(base, BPW)], idx_v)
        pltpu.sync_copy(embf_hbm, embf_v)
        pltpu.sync_copy(wb_hbm, wb_v)

        wrows = [wb_v[j] for j in range(6)]
        for j in range(6):
            bj = jnp.full((16,), wrows[j][8], jnp.float32)
            for i in range(4):
                val = embf_v[pl.ds(16 * i, 16)] * wrows[j][4 + i]
                if i == 0:
                    val = val + bj
                tab_v[j, pl.ds(16 * i, 16)] = val

        jcol = [jnp.full((16,), j, jnp.int32) for j in range(6)]
        ccol = [jnp.full((16,), c, jnp.int32) for c in range(4)]

        def chunk(t, carry):
            rows = lax.iota(jnp.int32, 16) + t * 16
            cc = [plsc.load_gather(cont_v, [rows, ccol[c]]) for c in range(4)]
            fidx = [plsc.load_gather(idx_v, [rows, ccol[i]]) + 16 * i
                    for i in range(4)]
            for j in range(6):
                acc = (cc[0] * wrows[j][0] + cc[1] * wrows[j][1]
                       + cc[2] * wrows[j][2] + cc[3] * wrows[j][3])
                for i in range(4):
                    acc = acc + plsc.load_gather(tab_v, [jcol[j], fidx[i]])
                plsc.store_scatter(out_v, [rows, jcol[j]], acc)
            return carry

        lax.fori_loop(0, CH, chunk, 0)
        pltpu.sync_copy(out_v, out_hbm.at[pl.ds(base, BPW)])

    return sc_encoder


def kernel(cont, cat_idx, emb, W, b):
    B = cont.shape[0]
    idx32 = cat_idx.astype(jnp.int32)
    embf = jnp.zeros((4, 16), jnp.float32).at[:, :10].set(emb[..., 0]).reshape(64)
    wb = jnp.zeros((6, 16), jnp.float32).at[:, :8].set(W).at[:, 8].set(b)
    return _build(B)(cont, idx32, embf, wb)


# speedup vs baseline: 1.0357x; 1.0357x over previous
import functools

import jax
import jax.numpy as jnp
from jax import lax
from jax.experimental import pallas as pl
from jax.experimental.pallas import tpu as pltpu
from jax.experimental.pallas import tpu_sc as plsc

NC = 2
NS = 16
L = 16
NW = NC * NS


@functools.cache
def _build(B: int):
    BPW = B // NW
    CH = BPW // L

    mesh = plsc.VectorSubcoreMesh(core_axis_name="c", subcore_axis_name="s",
                                  num_cores=NC, num_subcores=NS)

    @functools.partial(
        pl.kernel,
        out_type=jax.ShapeDtypeStruct((B, 6), jnp.float32),
        mesh=mesh,
        scratch_types=[
            pltpu.VMEM((BPW, 4), jnp.float32),
            pltpu.VMEM((BPW, 4), jnp.int32),
            pltpu.VMEM((BPW, 6), jnp.float32),
            pltpu.VMEM((48,), jnp.float32),
            pltpu.VMEM((48,), jnp.float32),
            pltpu.VMEM((16,), jnp.float32),
            pltpu.VMEM((6, 64), jnp.float32),
        ],
        compiler_params=pltpu.CompilerParams(
            needs_layout_passes=False, use_tc_tiling_on_sc=False),
    )
    def sc_encoder(cont_hbm, idx_hbm, emb_hbm, w_hbm, b_hbm, out_hbm,
                   cont_v, idx_v, out_v, emb_v, w_v, b_v, tab_v):
        wid = lax.axis_index("s") * NC + lax.axis_index("c")
        base = wid * BPW

        pltpu.sync_copy(cont_hbm.at[pl.ds(base, BPW)], cont_v)
        pltpu.sync_copy(idx_hbm.at[pl.ds(base, BPW)], idx_v)
        pltpu.sync_copy(emb_hbm, emb_v.at[pl.ds(0, 40)])
        pltpu.sync_copy(w_hbm, w_v)
        pltpu.sync_copy(b_hbm, b_v.at[pl.ds(0, 6)])

        lane = lax.iota(jnp.int32, 16)
        wch = [w_v[pl.ds(16 * r, 16)] for r in range(3)]
        bvec = b_v[...]

        def wscal(j, c):
            k = 8 * j + c
            return wch[k // 16][k % 16]

        ev = [plsc.load_gather(emb_v, [lane + 10 * i]) for i in range(4)]
        for j in range(6):
            bj = jnp.full((16,), bvec[j], jnp.float32)
            for i in range(4):
                val = ev[i] * wscal(j, 4 + i)
                if i == 0:
                    val = val + bj
                tab_v[j, pl.ds(16 * i, 16)] = val

        jcol = [jnp.full((16,), j, jnp.int32) for j in range(6)]
        ccol = [jnp.full((16,), c, jnp.int32) for c in range(4)]
        wd = [[wscal(j, c) for c in range(4)] for j in range(6)]

        def chunk(t, carry):
            rows = lax.iota(jnp.int32, 16) + t * 16
            cc = [plsc.load_gather(cont_v, [rows, ccol[c]]) for c in range(4)]
            fidx = [plsc.load_gather(idx_v, [rows, ccol[i]]) + 16 * i
                    for i in range(4)]
            for j in range(6):
                acc = (cc[0] * wd[j][0] + cc[1] * wd[j][1]
                       + cc[2] * wd[j][2] + cc[3] * wd[j][3])
                for i in range(4):
                    acc = acc + plsc.load_gather(tab_v, [jcol[j], fidx[i]])
                plsc.store_scatter(out_v, [rows, jcol[j]], acc)
            return carry

        lax.fori_loop(0, CH, chunk, 0)
        pltpu.sync_copy(out_v, out_hbm.at[pl.ds(base, BPW)])

    return sc_encoder


def kernel(cont, cat_idx, emb, W, b):
    B = cont.shape[0]
    idx32 = cat_idx.astype(jnp.int32)
    return _build(B)(cont, idx32, emb.reshape(40), W.reshape(48), b)


# speedup vs baseline: 1.1057x; 1.0676x over previous
import functools

import jax
import jax.numpy as jnp
from jax import lax
from jax.experimental import pallas as pl
from jax.experimental.pallas import tpu as pltpu
from jax.experimental.pallas import tpu_sc as plsc

NC = 2
NS = 16
L = 16
NW = NC * NS
UNROLL = 4


@functools.cache
def _build(B: int):
    BPW = B // NW
    CH = BPW // L

    mesh = plsc.VectorSubcoreMesh(core_axis_name="c", subcore_axis_name="s",
                                  num_cores=NC, num_subcores=NS)

    @functools.partial(
        pl.kernel,
        out_type=jax.ShapeDtypeStruct((B * 6,), jnp.float32),
        mesh=mesh,
        scratch_types=[
            pltpu.VMEM((BPW * 4,), jnp.float32),
            pltpu.VMEM((BPW * 4,), jnp.int32),
            pltpu.VMEM((BPW * 6,), jnp.float32),
            pltpu.VMEM((48,), jnp.float32),
            pltpu.VMEM((48,), jnp.float32),
            pltpu.VMEM((16,), jnp.float32),
            pltpu.VMEM((6, 64), jnp.float32),
        ],
        compiler_params=pltpu.CompilerParams(
            needs_layout_passes=False, use_tc_tiling_on_sc=False),
    )
    def sc_encoder(cont_hbm, idx_hbm, emb_hbm, w_hbm, b_hbm, out_hbm,
                   cont_v, idx_v, out_v, emb_v, w_v, b_v, tab_v):
        wid = lax.axis_index("s") * NC + lax.axis_index("c")
        base = wid * BPW

        pltpu.sync_copy(cont_hbm.at[pl.ds(base * 4, BPW * 4)], cont_v)
        pltpu.sync_copy(idx_hbm.at[pl.ds(base * 4, BPW * 4)], idx_v)
        pltpu.sync_copy(emb_hbm, emb_v.at[pl.ds(0, 40)])
        pltpu.sync_copy(w_hbm, w_v)
        pltpu.sync_copy(b_hbm, b_v.at[pl.ds(0, 6)])

        lane = lax.iota(jnp.int32, 16)
        wch = [w_v[pl.ds(16 * r, 16)] for r in range(3)]
        bvec = b_v[...]

        def wscal(j, c):
            k = 8 * j + c
            return wch[k // 16][k % 16]

        ev = [plsc.load_gather(emb_v, [lane + 10 * i]) for i in range(4)]
        for j in range(6):
            bj = jnp.full((16,), bvec[j], jnp.float32)
            for i in range(4):
                val = ev[i] * wscal(j, 4 + i)
                if i == 0:
                    val = val + bj
                tab_v[j, pl.ds(16 * i, 16)] = val

        wd = [[wscal(j, c) for c in range(4)] for j in range(6)]
        jcol = [jnp.full((16,), j, jnp.int32) for j in range(6)]
        lane4 = lane * 4
        lane6 = lane * 6

        def do_chunk(t):
            in_off = lane4 + t * 64
            out_off = lane6 + t * 96
            cc = [plsc.load_gather(cont_v, [in_off + c]) for c in range(4)]
            fidx = [plsc.load_gather(idx_v, [in_off + i]) + 16 * i
                    for i in range(4)]
            for j in range(6):
                acc = (cc[0] * wd[j][0] + cc[1] * wd[j][1]
                       + cc[2] * wd[j][2] + cc[3] * wd[j][3])
                for i in range(4):
                    acc = acc + plsc.load_gather(tab_v, [jcol[j], fidx[i]])
                plsc.store_scatter(out_v, [out_off + j], acc)

        def chunk_group(g, carry):
            for u in range(UNROLL):
                do_chunk(g * UNROLL + u)
            return carry

        lax.fori_loop(0, CH // UNROLL, chunk_group, 0)
        pltpu.sync_copy(out_v, out_hbm.at[pl.ds(base * 6, BPW * 6)])

    return sc_encoder


def kernel(cont, cat_idx, emb, W, b):
    B = cont.shape[0]
    idx32 = cat_idx.astype(jnp.int32)
    out = _build(B)(cont.reshape(B * 4), idx32.reshape(B * 4),
                    emb.reshape(40), W.reshape(48), b)
    return out.reshape(B, 6)


# speedup vs baseline: 1.6018x; 1.4487x over previous
import functools

import jax
import jax.numpy as jnp
from jax import lax
from jax.experimental import pallas as pl
from jax.experimental.pallas import tpu as pltpu
from jax.experimental.pallas import tpu_sc as plsc

NC = 2
NS = 16
L = 16
NW = NC * NS


@functools.cache
def _build(B: int):
    BPW = B // NW
    CH = BPW // L
    TB = BPW // 128

    mesh = plsc.VectorSubcoreMesh(core_axis_name="c", subcore_axis_name="s",
                                  num_cores=NC, num_subcores=NS)

    @functools.partial(
        pl.kernel,
        out_type=jax.ShapeDtypeStruct((B * 6,), jnp.float32),
        mesh=mesh,
        scratch_types=[
            pltpu.VMEM((TB, 4, 128), jnp.float32),
            pltpu.VMEM((TB, 4, 128), jnp.int32),
            pltpu.VMEM((BPW * 6,), jnp.float32),
            pltpu.VMEM((48,), jnp.float32),
            pltpu.VMEM((48,), jnp.float32),
            pltpu.VMEM((16,), jnp.float32),
            pltpu.VMEM((6, 64), jnp.float32),
        ],
        compiler_params=pltpu.CompilerParams(
            needs_layout_passes=False, use_tc_tiling_on_sc=False),
    )
    def sc_encoder(cont_hbm, idx_hbm, emb_hbm, w_hbm, b_hbm, out_hbm,
                   cont_v, idx_v, out_v, emb_v, w_v, b_v, tab_v):
        wid = lax.axis_index("s") * NC + lax.axis_index("c")
        base = wid * BPW

        pltpu.sync_copy(cont_hbm.at[pl.ds(wid * TB, TB)], cont_v)
        pltpu.sync_copy(idx_hbm.at[pl.ds(wid * TB, TB)], idx_v)
        pltpu.sync_copy(emb_hbm, emb_v.at[pl.ds(0, 40)])
        pltpu.sync_copy(w_hbm, w_v)
        pltpu.sync_copy(b_hbm, b_v.at[pl.ds(0, 6)])

        lane = lax.iota(jnp.int32, 16)
        wch = [w_v[pl.ds(16 * r, 16)] for r in range(3)]
        bvec = b_v[...]

        def wscal(j, c):
            k = 8 * j + c
            return wch[k // 16][k % 16]

        ev = [plsc.load_gather(emb_v, [lane + 10 * i]) for i in range(4)]
        for j in range(6):
            bj = jnp.full((16,), bvec[j], jnp.float32)
            for i in range(4):
                val = ev[i] * wscal(j, 4 + i)
                if i == 0:
                    val = val + bj
                tab_v[j, pl.ds(16 * i, 16)] = val

        wd = [[wscal(j, c) for c in range(4)] for j in range(6)]
        jcol = [jnp.full((16,), j, jnp.int32) for j in range(6)]
        lane6 = lane * 6

        for t in range(CH):
            blk, cc0 = t // 8, (16 * t) % 128
            cc = [cont_v[blk, c, pl.ds(cc0, 16)] for c in range(4)]
            fidx = [idx_v[blk, i, pl.ds(cc0, 16)] + 16 * i
                    for i in range(4)]
            for j in range(6):
                acc = (cc[0] * wd[j][0] + cc[1] * wd[j][1]
                       + cc[2] * wd[j][2] + cc[3] * wd[j][3])
                for i in range(4):
                    acc = acc + plsc.load_gather(tab_v, [jcol[j], fidx[i]])
                plsc.store_scatter(out_v, [lane6 + (96 * t + j)], acc)

        pltpu.sync_copy(out_v, out_hbm.at[pl.ds(base * 6, BPW * 6)])

    return sc_encoder


def kernel(cont, cat_idx, emb, W, b):
    B = cont.shape[0]
    idx32 = cat_idx.astype(jnp.int32)
    cont3 = cont.reshape(B // 128, 128, 4).transpose(0, 2, 1)
    idx3 = idx32.reshape(B // 128, 128, 4).transpose(0, 2, 1)
    out = _build(B)(cont3, idx3, emb.reshape(40), W.reshape(48), b)
    return out.reshape(B, 6)


# speedup vs baseline: 1.7720x; 1.1062x over previous
import functools

import jax
import jax.numpy as jnp
from jax import lax
from jax.experimental import pallas as pl
from jax.experimental.pallas import tpu as pltpu
from jax.experimental.pallas import tpu_sc as plsc

NC = 2
NS = 16
L = 16
NW = NC * NS


@functools.cache
def _build(B: int):
    BPW = B // NW
    CH = BPW // L
    TB = BPW // 128

    mesh = plsc.VectorSubcoreMesh(core_axis_name="c", subcore_axis_name="s",
                                  num_cores=NC, num_subcores=NS)

    @functools.partial(
        pl.kernel,
        out_type=jax.ShapeDtypeStruct((B, 6), jnp.float32),
        mesh=mesh,
        scratch_types=[
            pltpu.VMEM((TB, 4, 128), jnp.float32),
            pltpu.VMEM((TB, 4, 128), jnp.int32),
            pltpu.VMEM((BPW, 6), jnp.float32),
            pltpu.VMEM((96,), jnp.float32),
            pltpu.VMEM((6, 64), jnp.float32),
        ],
        compiler_params=pltpu.CompilerParams(
            needs_layout_passes=False, use_tc_tiling_on_sc=False),
    )
    def sc_encoder(cont_hbm, idx_hbm, wts_hbm, out_hbm,
                   cont_v, idx_v, out_v, wts_v, tab_v):
        wid = lax.axis_index("s") * NC + lax.axis_index("c")
        base = wid * BPW

        pltpu.sync_copy(cont_hbm.at[pl.ds(wid * TB, TB)], cont_v)
        pltpu.sync_copy(idx_hbm.at[pl.ds(wid * TB, TB)], idx_v)
        pltpu.sync_copy(wts_hbm, wts_v)

        lane = lax.iota(jnp.int32, 16)
        wch = [wts_v[pl.ds(40 + 16 * r, 16)] for r in range(3)]
        bvec = wts_v[pl.ds(80, 16)]

        def wscal(j, c):
            k = 8 * j + c
            return wch[k // 16][k % 16]

        ev = [plsc.load_gather(wts_v, [lane + 10 * i]) for i in range(4)]
        for j in range(6):
            bj = jnp.full((16,), bvec[8 + j], jnp.float32)
            for i in range(4):
                val = ev[i] * wscal(j, 4 + i)
                if i == 0:
                    val = val + bj
                tab_v[j, pl.ds(16 * i, 16)] = val

        wd = [[wscal(j, c) for c in range(4)] for j in range(6)]
        jcol = [jnp.full((16,), j, jnp.int32) for j in range(6)]

        for t in range(CH):
            blk, cc0 = t // 8, (16 * t) % 128
            rows = lane + 16 * t
            cc = [cont_v[blk, c, pl.ds(cc0, 16)] for c in range(4)]
            fidx = [idx_v[blk, i, pl.ds(cc0, 16)] + 16 * i
                    for i in range(4)]
            for j in range(6):
                acc = (cc[0] * wd[j][0] + cc[1] * wd[j][1]
                       + cc[2] * wd[j][2] + cc[3] * wd[j][3])
                for i in range(4):
                    acc = acc + plsc.load_gather(tab_v, [jcol[j], fidx[i]])
                plsc.store_scatter(out_v, [rows, jcol[j]], acc)

        pltpu.sync_copy(out_v, out_hbm.at[pl.ds(base, BPW)])

    return sc_encoder


def kernel(cont, cat_idx, emb, W, b):
    B = cont.shape[0]
    idx32 = cat_idx.astype(jnp.int32)
    cont3 = cont.reshape(B // 128, 128, 4).transpose(0, 2, 1)
    idx3 = idx32.reshape(B // 128, 128, 4).transpose(0, 2, 1)
    wts = jnp.concatenate(
        [emb.reshape(40), W.reshape(48), b, jnp.zeros((2,), jnp.float32)])
    return _build(B)(cont3, idx3, wts)


# speedup vs baseline: 2.5930x; 1.4633x over previous
import functools

import jax
import jax.numpy as jnp
from jax import lax
from jax.experimental import pallas as pl
from jax.experimental.pallas import tpu as pltpu
from jax.experimental.pallas import tpu_sc as plsc

NC = 2
NS = 16
L = 16
NW = NC * NS


@functools.cache
def _build(B: int):
    BPW = B // NW
    CH = BPW // L
    TB = BPW // 128
    NB = B // 128

    mesh = plsc.VectorSubcoreMesh(core_axis_name="c", subcore_axis_name="s",
                                  num_cores=NC, num_subcores=NS)

    @functools.partial(
        pl.kernel,
        out_type=jax.ShapeDtypeStruct((NB, 8, 128), jnp.float32),
        mesh=mesh,
        scratch_types=[
            pltpu.VMEM((TB, 4, 128), jnp.float32),
            pltpu.VMEM((TB, 4, 128), jnp.int32),
            pltpu.VMEM((TB, 8, 128), jnp.float32),
            pltpu.VMEM((48,), jnp.float32),
            pltpu.VMEM((48,), jnp.float32),
            pltpu.VMEM((16,), jnp.float32),
            pltpu.VMEM((6, 64), jnp.float32),
        ],
        compiler_params=pltpu.CompilerParams(
            needs_layout_passes=False, use_tc_tiling_on_sc=False),
    )
    def sc_encoder(cont_hbm, idx_hbm, emb_hbm, w_hbm, b_hbm, out_hbm,
                   cont_v, idx_v, out_v, emb_v, w_v, b_v, tab_v):
        wid = lax.axis_index("s") * NC + lax.axis_index("c")

        pltpu.sync_copy(cont_hbm.at[pl.ds(wid * TB, TB)], cont_v)
        pltpu.sync_copy(idx_hbm.at[pl.ds(wid * TB, TB)], idx_v)
        pltpu.sync_copy(emb_hbm, emb_v.at[pl.ds(0, 40)])
        pltpu.sync_copy(w_hbm, w_v)
        pltpu.sync_copy(b_hbm, b_v.at[pl.ds(0, 6)])

        lane = lax.iota(jnp.int32, 16)
        wch = [w_v[pl.ds(16 * r, 16)] for r in range(3)]
        bvec = b_v[...]

        def wscal(j, c):
            k = 8 * j + c
            return wch[k // 16][k % 16]

        ev = [plsc.load_gather(emb_v, [lane + 10 * i]) for i in range(4)]
        for j in range(6):
            bj = jnp.full((16,), bvec[j], jnp.float32)
            for i in range(4):
                val = ev[i] * wscal(j, 4 + i)
                if i == 0:
                    val = val + bj
                tab_v[j, pl.ds(16 * i, 16)] = val

        wd = [[wscal(j, c) for c in range(4)] for j in range(6)]
        jcol = [jnp.full((16,), j, jnp.int32) for j in range(6)]

        for t in range(CH):
            blk, cc0 = t // 8, (16 * t) % 128
            cc = [cont_v[blk, c, pl.ds(cc0, 16)] for c in range(4)]
            fidx = [idx_v[blk, i, pl.ds(cc0, 16)] + 16 * i
                    for i in range(4)]
            for j in range(6):
                acc = (cc[0] * wd[j][0] + cc[1] * wd[j][1]
                       + cc[2] * wd[j][2] + cc[3] * wd[j][3])
                acc = acc + ((plsc.load_gather(tab_v, [jcol[j], fidx[0]])
                              + plsc.load_gather(tab_v, [jcol[j], fidx[1]]))
                             + (plsc.load_gather(tab_v, [jcol[j], fidx[2]])
                                + plsc.load_gather(tab_v, [jcol[j], fidx[3]])))
                out_v[blk, j, pl.ds(cc0, 16)] = acc

        pltpu.sync_copy(out_v, out_hbm.at[pl.ds(wid * TB, TB)])

    return sc_encoder


def kernel(cont, cat_idx, emb, W, b):
    B = cont.shape[0]
    idx32 = cat_idx.astype(jnp.int32)
    cont3 = cont.reshape(B // 128, 128, 4).transpose(0, 2, 1)
    idx3 = idx32.reshape(B // 128, 128, 4).transpose(0, 2, 1)
    out3 = _build(B)(cont3, idx3, emb.reshape(40), W.reshape(48), b)
    return out3.transpose(0, 2, 1).reshape(B, 8)[:, :6]
